# baseline (speedup 1.0000x reference)
import functools
import math

import jax
import jax.numpy as jnp
from jax import lax
from jax.experimental import pallas as pl
from jax.experimental.pallas import tpu as pltpu
from jax.experimental.pallas import tpu_sc as plsc

_D_MODEL = 64
_SCALE = math.sqrt(_D_MODEL)
_LANES = 16


@functools.lru_cache(maxsize=None)
def _build(n_rows: int, d: int, chunk: int):
    info = plsc.get_sparse_core_info()
    nw = info.num_cores * info.num_subcores
    assert n_rows % (nw * chunk) == 0
    b_per_w = n_rows // nw
    n_chunks = b_per_w // chunk
    mesh = plsc.VectorSubcoreMesh(core_axis_name="c", subcore_axis_name="s")

    @functools.partial(
        pl.kernel,
        out_type=jax.ShapeDtypeStruct((n_rows, d), jnp.float32),
        mesh=mesh,
        scratch_types=[
            pltpu.VMEM((chunk,), jnp.int32),
            pltpu.VMEM((chunk, d), jnp.float32),
            pltpu.SemaphoreType.DMA,
        ],
        compiler_params=pltpu.CompilerParams(use_tc_tiling_on_sc=False),
    )
    def emb_kernel(idx_hbm, table_hbm, out_hbm, idx_v, rows_v, sem):
        wid = lax.axis_index("s") * info.num_cores + lax.axis_index("c")
        base = wid * b_per_w

        def chunk_body(ci, carry):
            off = base + ci * chunk
            pltpu.sync_copy(idx_hbm.at[pl.ds(off, chunk)], idx_v)
            pltpu.async_copy(table_hbm.at[idx_v], rows_v, sem).wait()

            def row_body(r, c):
                for j in range(d // _LANES):
                    s = pl.ds(j * _LANES, _LANES)
                    rows_v[r, s] = rows_v[r, s] * _SCALE
                return c

            lax.fori_loop(0, chunk, row_body, 0, unroll=4)
            pltpu.sync_copy(rows_v, out_hbm.at[pl.ds(off, chunk)])
            return carry

        lax.fori_loop(0, n_chunks, chunk_body, 0)

    return emb_kernel


def kernel(x, table):
    b, s = x.shape
    d = table.shape[1]
    n_rows = b * s
    flat_idx = x.reshape(n_rows)
    out = _build(n_rows, d, 1024)(flat_idx, table)
    return out.reshape(b, s, d)


# speedup vs baseline: 1.0547x; 1.0547x over previous
import functools
import math

import jax
import jax.numpy as jnp
from jax import lax
from jax.experimental import pallas as pl
from jax.experimental.pallas import tpu as pltpu
from jax.experimental.pallas import tpu_sc as plsc

_D_MODEL = 64
_SCALE = math.sqrt(_D_MODEL)
_LANES = 16


@functools.lru_cache(maxsize=None)
def _build(n_rows: int, d: int, chunk: int):
    info = plsc.get_sparse_core_info()
    nw = info.num_cores * info.num_subcores
    assert n_rows % (nw * chunk) == 0
    b_per_w = n_rows // nw
    n_chunks = b_per_w // chunk
    assert n_chunks % 2 == 0 and n_chunks >= 4
    mesh = plsc.VectorSubcoreMesh(core_axis_name="c", subcore_axis_name="s")

    @functools.partial(
        pl.kernel,
        out_type=jax.ShapeDtypeStruct((n_rows, d), jnp.float32),
        mesh=mesh,
        scratch_types=[
            pltpu.VMEM((chunk,), jnp.int32),
            pltpu.VMEM((chunk,), jnp.int32),
            pltpu.VMEM((chunk, d), jnp.float32),
            pltpu.VMEM((chunk, d), jnp.float32),
            pltpu.SemaphoreType.DMA,
            pltpu.SemaphoreType.DMA,
        ],
        compiler_params=pltpu.CompilerParams(use_tc_tiling_on_sc=False),
    )
    def emb_kernel(idx_hbm, table_hbm, out_hbm, idx0, idx1, rows0, rows1,
                   sem0, sem1):
        wid = lax.axis_index("s") * info.num_cores + lax.axis_index("c")
        base = wid * b_per_w
        idxs = (idx0, idx1)
        rowss = (rows0, rows1)
        sems = (sem0, sem1)

        def start_gather(ci, b):
            off = base + ci * chunk
            pltpu.sync_copy(idx_hbm.at[pl.ds(off, chunk)], idxs[b])
            pltpu.async_copy(table_hbm.at[idxs[b]], rowss[b], sems[b])

        def finish(ci, b):
            pltpu.make_async_copy(table_hbm.at[idxs[b]], rowss[b],
                                  sems[b]).wait()
            rv = rowss[b]

            def row_body(r, c):
                for j in range(d // _LANES):
                    s = pl.ds(j * _LANES, _LANES)
                    rv[r, s] = rv[r, s] * _SCALE
                return c

            lax.fori_loop(0, chunk, row_body, 0, unroll=4)
            off = base + ci * chunk
            pltpu.sync_copy(rv, out_hbm.at[pl.ds(off, chunk)])

        start_gather(0, 0)

        def pair_body(i, carry):
            g = 2 * i
            start_gather(g + 1, 1)
            finish(g, 0)
            start_gather(g + 2, 0)
            finish(g + 1, 1)
            return carry

        lax.fori_loop(0, n_chunks // 2 - 1, pair_body, 0)
        g = n_chunks - 2
        start_gather(g + 1, 1)
        finish(g, 0)
        finish(g + 1, 1)

    return emb_kernel


def kernel(x, table):
    b, s = x.shape
    d = table.shape[1]
    n_rows = b * s
    flat_idx = x.reshape(n_rows)
    out = _build(n_rows, d, 800)(flat_idx, table)
    return out.reshape(b, s, d)


# speedup vs baseline: 1.0565x; 1.0016x over previous
import functools
import math

import jax
import jax.numpy as jnp
from jax import lax
from jax.experimental import pallas as pl
from jax.experimental.pallas import tpu as pltpu
from jax.experimental.pallas import tpu_sc as plsc

_D_MODEL = 64
_SCALE = math.sqrt(_D_MODEL)
_LANES = 16


@functools.lru_cache(maxsize=None)
def _build(n_rows: int, d: int, chunk: int):
    info = plsc.get_sparse_core_info()
    nw = info.num_cores * info.num_subcores
    assert n_rows % (nw * chunk) == 0
    b_per_w = n_rows // nw
    n_chunks = b_per_w // chunk
    assert n_chunks % 2 == 0 and n_chunks >= 4
    mesh = plsc.VectorSubcoreMesh(core_axis_name="c", subcore_axis_name="s")

    @functools.partial(
        pl.kernel,
        out_type=jax.ShapeDtypeStruct((n_rows, d), jnp.float32),
        mesh=mesh,
        scratch_types=[
            pltpu.VMEM((chunk,), jnp.int32),
            pltpu.VMEM((chunk,), jnp.int32),
            pltpu.VMEM((chunk, d), jnp.float32),
            pltpu.VMEM((chunk, d), jnp.float32),
            pltpu.SemaphoreType.DMA,
            pltpu.SemaphoreType.DMA,
        ],
        compiler_params=pltpu.CompilerParams(use_tc_tiling_on_sc=False),
    )
    def emb_kernel(idx_hbm, table_hbm, out_hbm, idx0, idx1, rows0, rows1,
                   sem0, sem1):
        wid = lax.axis_index("s") * info.num_cores + lax.axis_index("c")
        base = wid * b_per_w
        idxs = (idx0, idx1)
        rowss = (rows0, rows1)
        sems = (sem0, sem1)

        def start_gather(ci, b):
            off = base + ci * chunk
            pltpu.sync_copy(idx_hbm.at[pl.ds(off, chunk)], idxs[b])
            pltpu.async_copy(table_hbm.at[idxs[b]], rowss[b], sems[b])

        def finish(ci, b):
            pltpu.make_async_copy(table_hbm.at[idxs[b]], rowss[b],
                                  sems[b]).wait()
            rv = rowss[b]

            @plsc.parallel_loop(0, chunk, unroll=8)
            def row_body(r):
                for j in range(d // _LANES):
                    s = pl.ds(j * _LANES, _LANES)
                    rv[r, s] = rv[r, s] * _SCALE
            off = base + ci * chunk
            pltpu.sync_copy(rv, out_hbm.at[pl.ds(off, chunk)])

        start_gather(0, 0)

        def pair_body(i, carry):
            g = 2 * i
            start_gather(g + 1, 1)
            finish(g, 0)
            start_gather(g + 2, 0)
            finish(g + 1, 1)
            return carry

        lax.fori_loop(0, n_chunks // 2 - 1, pair_body, 0)
        g = n_chunks - 2
        start_gather(g + 1, 1)
        finish(g, 0)
        finish(g + 1, 1)

    return emb_kernel


def kernel(x, table):
    b, s = x.shape
    d = table.shape[1]
    n_rows = b * s
    flat_idx = x.reshape(n_rows)
    out = _build(n_rows, d, 800)(flat_idx, table)
    return out.reshape(b, s, d)
